# baseline (speedup 1.0000x reference)
import functools
import jax
import jax.numpy as jnp
from jax import lax
from jax.experimental import pallas as pl
from jax.experimental.pallas import tpu as pltpu
from jax.experimental.pallas import tpu_sc as plsc

_H = 32
_NH = 12
_S = _H * _H
_NW = 32
_PW = _S // _NW


def _sc_body(nb, tf_hbm, x_hbm, out_hbm, table_v, bias_v, xbuf_v):
    w = lax.axis_index("s") * 2 + lax.axis_index("c")

    def h_body(h, carry):
        pltpu.sync_copy(tf_hbm.at[h], table_v)

        def build_body(pj, c):
            for qi in range(_H):
                a = w + (_H - 1) - qi
                bias_v[pj, pl.ds(qi * _H, 16)] = table_v[a, pl.ds(_H - 1 - pj, 16)]
                bias_v[pj, pl.ds(qi * _H + 16, 16)] = table_v[a, pl.ds(_H - 1 - pj + 16, 16)]
            return c

        lax.fori_loop(0, _H, build_body, 0)

        for b in range(nb):
            pltpu.sync_copy(x_hbm.at[b, h, pl.ds(w * _PW, _PW)], xbuf_v)

            def add_body(row, c):
                for cc in range(_S // 16):
                    sl = pl.ds(cc * 16, 16)
                    xbuf_v[row, sl] = xbuf_v[row, sl] + bias_v[row, sl]
                return c

            lax.fori_loop(0, _PW, add_body, 0)
            pltpu.sync_copy(xbuf_v, out_hbm.at[b, h, pl.ds(w * _PW, _PW)])
        return carry

    lax.fori_loop(0, _NH, h_body, 0)


def kernel(x, relative_pos):
    nb = x.shape[0]
    tf = jnp.pad(relative_pos[:, :, ::-1], ((0, 0), (0, 1), (0, 1)))
    mesh = plsc.VectorSubcoreMesh(core_axis_name="c", subcore_axis_name="s",
                                  num_cores=2)
    run = pl.kernel(
        functools.partial(_sc_body, nb),
        mesh=mesh,
        out_type=jax.ShapeDtypeStruct(x.shape, x.dtype),
        scratch_types=[
            pltpu.VMEM((64, 64), jnp.float32),
            pltpu.VMEM((_PW, _S), jnp.float32),
            pltpu.VMEM((_PW, _S), jnp.float32),
        ],
    )
    return run(tf, x)
